# baseline (speedup 1.0000x reference)
import functools

import jax
import jax.numpy as jnp
from jax import lax
from jax.experimental import pallas as pl
from jax.experimental.pallas import tpu as pltpu
from jax.experimental.pallas import tpu_sc as plsc

N = 10000
E = 320000
D = 128

NC = 2
NS = 16
NW = NC * NS
K = 128
STEPS = 80
TOT = NW * STEPS
EP = TOT * K
S_A = 120
S_B = 40
PH = 40
NP = 10240
RPS = NP // NS
PPS = RPS // 8


def _deg_body(dst_hbm, ones_hbm, zeros_hbm, out_hbm, idx_v, ones_v, deg_sp, sem):
    cid = lax.axis_index("c")
    sid = lax.axis_index("s")
    g = cid * NS + sid
    pltpu.sync_copy(zeros_hbm.at[pl.ds(sid * RPS, RPS)],
                    deg_sp.at[pl.ds(sid * RPS, RPS)])
    pltpu.sync_copy(ones_hbm, ones_v)
    pltpu.sync_copy(dst_hbm.at[pl.ds(g * STEPS, STEPS)], idx_v)
    plsc.subcore_barrier()

    @pl.loop(0, STEPS)
    def _(j):
        pltpu.sync_copy(ones_v, deg_sp.at[idx_v.at[j]], add=True)

    plsc.subcore_barrier()
    pltpu.sync_copy(deg_sp.at[pl.ds(sid * RPS, RPS)],
                    out_hbm.at[cid, pl.ds(sid * RPS, RPS)])


def _edge_body(hs_hbm, src_hbm, dst_hbm, zeros_hbm, out_hbm,
               sidx_v, didx_v, rows0, rows1, acc_sp, gsem0, gsem1):
    cid = lax.axis_index("c")
    sid = lax.axis_index("s")
    pltpu.sync_copy(zeros_hbm.at[pl.ds(sid * RPS, RPS)],
                    acc_sp.at[pl.ds(sid * RPS, RPS)])
    plsc.subcore_barrier()

    base = jnp.where(cid == 0, sid * S_A, NS * S_A + sid * S_B)
    nph = jnp.where(cid == 0, S_A // PH, S_B // PH)

    @pl.loop(0, nph)
    def _(ph):
        pltpu.sync_copy(src_hbm.at[pl.ds(base + ph * PH, PH)], sidx_v)
        pltpu.sync_copy(dst_hbm.at[pl.ds(base + ph * PH, PH)], didx_v)

        @pl.loop(0, PH, step=2)
        def _(j):
            da = pltpu.async_copy(hs_hbm.at[sidx_v.at[j]], rows0, gsem0)
            db = pltpu.async_copy(hs_hbm.at[sidx_v.at[j + 1]], rows1, gsem1)
            da.wait()
            pltpu.sync_copy(rows0, acc_sp.at[didx_v.at[j]], add=True)
            db.wait()
            pltpu.sync_copy(rows1, acc_sp.at[didx_v.at[j + 1]], add=True)

    plsc.subcore_barrier()
    pltpu.sync_copy(acc_sp.at[pl.ds(sid * RPS, RPS)],
                    out_hbm.at[cid, pl.ds(sid * RPS, RPS)])


def _gonly_body(hs_hbm, src_hbm, out_hbm, sidx_v, rows0, rows1, gsem0, gsem1):
    cid = lax.axis_index("c")
    sid = lax.axis_index("s")
    g = cid * NS + sid
    pltpu.sync_copy(src_hbm.at[pl.ds(g * STEPS, STEPS)], sidx_v)

    @pl.loop(0, STEPS, step=2)
    def _(j):
        da = pltpu.async_copy(hs_hbm.at[sidx_v.at[j]], rows0, gsem0)
        db = pltpu.async_copy(hs_hbm.at[sidx_v.at[j + 1]], rows1, gsem1)
        da.wait()
        db.wait()

    pltpu.sync_copy(rows0, out_hbm.at[cid, pl.ds(sid * K, K)])


@functools.cache
def _sc_kernels():
    mesh = plsc.VectorSubcoreMesh(core_axis_name="c", subcore_axis_name="s",
                                  num_cores=NC, num_subcores=NS)
    deg_kernel = pl.kernel(
        _deg_body,
        out_type=jax.ShapeDtypeStruct((NC, NP, D), jnp.float32),
        mesh=mesh,
        scratch_types=[
            pltpu.VMEM((STEPS, K), jnp.int32),
            pltpu.VMEM((K, D), jnp.float32),
            pltpu.VMEM_SHARED((NP, D), jnp.float32),
            pltpu.SemaphoreType.DMA,
        ],
    )
    edge_kernel = pl.kernel(
        _edge_body,
        out_type=jax.ShapeDtypeStruct((NC, NP, D), jnp.float32),
        mesh=mesh,
        scratch_types=[
            pltpu.VMEM((PH, K), jnp.int32),
            pltpu.VMEM((PH, K), jnp.int32),
            pltpu.VMEM((K, D), jnp.float32),
            pltpu.VMEM((K, D), jnp.float32),
            pltpu.VMEM_SHARED((NP, D), jnp.float32),
            pltpu.SemaphoreType.DMA,
            pltpu.SemaphoreType.DMA,
        ],
    )
    gonly_kernel = pl.kernel(
        _gonly_body,
        out_type=jax.ShapeDtypeStruct((NC, NP, D), jnp.float32),
        mesh=mesh,
        scratch_types=[
            pltpu.VMEM((STEPS, K), jnp.int32),
            pltpu.VMEM((K, D), jnp.float32),
            pltpu.VMEM((K, D), jnp.float32),
            pltpu.SemaphoreType.DMA,
            pltpu.SemaphoreType.DMA,
        ],
    )
    return deg_kernel, edge_kernel, gonly_kernel


BH = 640


def _hs_body(x_ref, w_ref, d0_ref, d1_ref, hs_ref):
    deg = d0_ref[...] + d1_ref[...] + 1.0
    dinv = lax.rsqrt(deg)
    hs_ref[...] = jnp.dot(x_ref[...], w_ref[...],
                          preferred_element_type=jnp.float32) * dinv


_hs_call = pl.pallas_call(
    _hs_body,
    grid=(NP // BH,),
    in_specs=[
        pl.BlockSpec((BH, D), lambda i: (i, 0)),
        pl.BlockSpec((D, D), lambda i: (0, 0)),
        pl.BlockSpec((BH, 1), lambda i: (i, 0)),
        pl.BlockSpec((BH, 1), lambda i: (i, 0)),
    ],
    out_specs=pl.BlockSpec((BH, D), lambda i: (i, 0)),
    out_shape=jax.ShapeDtypeStruct((NP, D), jnp.float32),
)


def _out_body(acc0_ref, acc1_ref, hs_ref, d0_ref, d1_ref, b_ref, o_ref):
    deg = d0_ref[...] + d1_ref[...] + 1.0
    dinv = lax.rsqrt(deg)
    o_ref[...] = dinv * (acc0_ref[...] + acc1_ref[...] + hs_ref[...]) + b_ref[...]


_out_call = pl.pallas_call(
    _out_body,
    grid=(NP // BH,),
    in_specs=[
        pl.BlockSpec((BH, D), lambda i: (i, 0)),
        pl.BlockSpec((BH, D), lambda i: (i, 0)),
        pl.BlockSpec((BH, D), lambda i: (i, 0)),
        pl.BlockSpec((BH, 1), lambda i: (i, 0)),
        pl.BlockSpec((BH, 1), lambda i: (i, 0)),
        pl.BlockSpec((1, D), lambda i: (0, 0)),
    ],
    out_specs=pl.BlockSpec((BH, D), lambda i: (i, 0)),
    out_shape=jax.ShapeDtypeStruct((NP, D), jnp.float32),
)


def kernel(x, edge_index, W, b):
    src = edge_index[0].astype(jnp.int32)
    dst = edge_index[1].astype(jnp.int32)
    pad = jnp.full((EP - E,), N, jnp.int32)
    src3 = jnp.concatenate([src, pad]).reshape(TOT, K)
    dst3 = jnp.concatenate([dst, pad]).reshape(TOT, K)

    ones128 = jnp.ones((K, D), jnp.float32)
    zerosD = jnp.zeros((NP, D), jnp.float32)

    deg_kernel, edge_kernel, gonly_kernel = _sc_kernels()
    degp = deg_kernel(dst3, ones128, zerosD)
    d0 = degp[0][:, 0:1]
    d1 = degp[1][:, 0:1]

    x_pad = jnp.concatenate([x, jnp.zeros((NP - N, D), x.dtype)])
    hs = _hs_call(x_pad, W, d0, d1)

    g1 = gonly_kernel(hs, src3)
    g2 = gonly_kernel(x_pad + g1[0, 0, 0] * 0.0, src3)
    accp = edge_kernel(hs + g2[0, 0, 0] * 0.0, src3, dst3, zerosD)

    out_pad = _out_call(accp[0], accp[1], hs, d0, d1, b.reshape(1, D))
    return out_pad[:N]


# speedup vs baseline: 2.0670x; 2.0670x over previous
import functools

import jax
import jax.numpy as jnp
from jax import lax
from jax.experimental import pallas as pl
from jax.experimental.pallas import tpu as pltpu
from jax.experimental.pallas import tpu_sc as plsc

N = 10000
E = 320000
D = 128

NC = 2
NS = 16
NW = NC * NS
K = 128
STEPS = 80
TOT = NW * STEPS
EP = TOT * K
SE = 160
PH = 40
NPH = SE // PH
NP = 10240
RPS = NP // NS
PPS = RPS // 8


def _deg_body(dst_hbm, ones_hbm, zeros_hbm, out_hbm, idx_v, ones_v, deg_sp, sem):
    cid = lax.axis_index("c")
    sid = lax.axis_index("s")
    g = cid * NS + sid
    pltpu.sync_copy(zeros_hbm.at[pl.ds(sid * RPS, RPS)],
                    deg_sp.at[pl.ds(sid * RPS, RPS)])
    pltpu.sync_copy(ones_hbm, ones_v)
    pltpu.sync_copy(dst_hbm.at[pl.ds(g * STEPS, STEPS)], idx_v)
    plsc.subcore_barrier()

    @pl.loop(0, STEPS)
    def _(j):
        pltpu.sync_copy(ones_v, deg_sp.at[idx_v.at[j]], add=True)

    plsc.subcore_barrier()
    pltpu.sync_copy(deg_sp.at[pl.ds(sid * RPS, RPS)],
                    out_hbm.at[cid, pl.ds(sid * RPS, RPS)])


def _edge_body(hs_hbm, src_hbm, dst_hbm, zeros_hbm, out_hbm,
               sidx_v, didx_v, rows0, rows1, acc_sp, gsem0, gsem1):
    cid = lax.axis_index("c")
    sid = lax.axis_index("s")

    @pl.when(cid == 0)
    def _():
        pltpu.sync_copy(zeros_hbm.at[pl.ds(sid * RPS, RPS)],
                        acc_sp.at[pl.ds(sid * RPS, RPS)])
        plsc.subcore_barrier()
        base = sid * SE

        @pl.loop(0, NPH)
        def _(ph):
            pltpu.sync_copy(src_hbm.at[pl.ds(base + ph * PH, PH)], sidx_v)
            pltpu.sync_copy(dst_hbm.at[pl.ds(base + ph * PH, PH)], didx_v)

            @pl.loop(0, PH, step=2)
            def _(j):
                da = pltpu.async_copy(hs_hbm.at[sidx_v.at[j]], rows0, gsem0)
                db = pltpu.async_copy(hs_hbm.at[sidx_v.at[j + 1]], rows1, gsem1)
                da.wait()
                pltpu.sync_copy(rows0, acc_sp.at[didx_v.at[j]], add=True)
                db.wait()
                pltpu.sync_copy(rows1, acc_sp.at[didx_v.at[j + 1]], add=True)

        plsc.subcore_barrier()
        pltpu.sync_copy(acc_sp.at[pl.ds(sid * RPS, RPS)],
                        out_hbm.at[pl.ds(sid * RPS, RPS)])


@functools.cache
def _sc_kernels():
    mesh = plsc.VectorSubcoreMesh(core_axis_name="c", subcore_axis_name="s",
                                  num_cores=NC, num_subcores=NS)
    deg_kernel = pl.kernel(
        _deg_body,
        out_type=jax.ShapeDtypeStruct((NC, NP, D), jnp.float32),
        mesh=mesh,
        scratch_types=[
            pltpu.VMEM((STEPS, K), jnp.int32),
            pltpu.VMEM((K, D), jnp.float32),
            pltpu.VMEM_SHARED((NP, D), jnp.float32),
            pltpu.SemaphoreType.DMA,
        ],
    )
    edge_kernel = pl.kernel(
        _edge_body,
        out_type=jax.ShapeDtypeStruct((NP, D), jnp.float32),
        mesh=mesh,
        scratch_types=[
            pltpu.VMEM((PH, K), jnp.int32),
            pltpu.VMEM((PH, K), jnp.int32),
            pltpu.VMEM((K, D), jnp.float32),
            pltpu.VMEM((K, D), jnp.float32),
            pltpu.VMEM_SHARED((NP, D), jnp.float32),
            pltpu.SemaphoreType.DMA,
            pltpu.SemaphoreType.DMA,
        ],
    )
    return deg_kernel, edge_kernel


BH = 640


def _hs_body(x_ref, w_ref, d0_ref, d1_ref, hs_ref):
    deg = d0_ref[...] + d1_ref[...] + 1.0
    dinv = lax.rsqrt(deg)
    hs_ref[...] = jnp.dot(x_ref[...], w_ref[...],
                          preferred_element_type=jnp.float32) * dinv


_hs_call = pl.pallas_call(
    _hs_body,
    grid=(NP // BH,),
    in_specs=[
        pl.BlockSpec((BH, D), lambda i: (i, 0)),
        pl.BlockSpec((D, D), lambda i: (0, 0)),
        pl.BlockSpec((BH, 1), lambda i: (i, 0)),
        pl.BlockSpec((BH, 1), lambda i: (i, 0)),
    ],
    out_specs=pl.BlockSpec((BH, D), lambda i: (i, 0)),
    out_shape=jax.ShapeDtypeStruct((NP, D), jnp.float32),
)


def _out_body(acc_ref, hs_ref, d0_ref, d1_ref, b_ref, o_ref):
    deg = d0_ref[...] + d1_ref[...] + 1.0
    dinv = lax.rsqrt(deg)
    o_ref[...] = dinv * (acc_ref[...] + hs_ref[...]) + b_ref[...]


_out_call = pl.pallas_call(
    _out_body,
    grid=(NP // BH,),
    in_specs=[
        pl.BlockSpec((BH, D), lambda i: (i, 0)),
        pl.BlockSpec((BH, D), lambda i: (i, 0)),
        pl.BlockSpec((BH, 1), lambda i: (i, 0)),
        pl.BlockSpec((BH, 1), lambda i: (i, 0)),
        pl.BlockSpec((1, D), lambda i: (0, 0)),
    ],
    out_specs=pl.BlockSpec((BH, D), lambda i: (i, 0)),
    out_shape=jax.ShapeDtypeStruct((N, D), jnp.float32),
)


def kernel(x, edge_index, W, b):
    src = edge_index[0].astype(jnp.int32)
    dst = edge_index[1].astype(jnp.int32)
    pad = jnp.full((EP - E,), N, jnp.int32)
    src3 = jnp.concatenate([src, pad]).reshape(TOT, K)
    dst3 = jnp.concatenate([dst, pad]).reshape(TOT, K)

    ones128 = jnp.ones((K, D), jnp.float32)
    zerosD = jnp.zeros((NP, D), jnp.float32)

    deg_kernel, edge_kernel = _sc_kernels()
    degp = deg_kernel(dst3, ones128, zerosD)
    d0 = degp[0][:, 0:1]
    d1 = degp[1][:, 0:1]

    x_pad = jnp.concatenate([x, jnp.zeros((NP - N, D), x.dtype)])
    hs = _hs_call(x_pad, W, d0, d1)

    acc = edge_kernel(hs, src3, dst3, zerosD)

    return _out_call(acc, hs, d0, d1, b.reshape(1, D))


# speedup vs baseline: 2.4300x; 1.1756x over previous
import functools

import jax
import jax.numpy as jnp
from jax import lax
from jax.experimental import pallas as pl
from jax.experimental.pallas import tpu as pltpu
from jax.experimental.pallas import tpu_sc as plsc

N = 10000
E = 320000
D = 128

NC = 2
NS = 16
NW = NC * NS
K = 128
STEPS = 80
TOT = NW * STEPS
EP = TOT * K
S_A = 120
S_B = 40
PH = 40
NP = 10240
RPS = NP // NS
PPS = RPS // 8


def _deg_body(dst_hbm, ones_hbm, zeros_hbm, out_hbm, idx_v, ones_v, deg_sp, sem):
    cid = lax.axis_index("c")
    sid = lax.axis_index("s")
    g = cid * NS + sid
    pltpu.sync_copy(zeros_hbm.at[pl.ds(sid * RPS, RPS)],
                    deg_sp.at[pl.ds(sid * RPS, RPS)])
    pltpu.sync_copy(ones_hbm, ones_v)
    pltpu.sync_copy(dst_hbm.at[pl.ds(g * STEPS, STEPS)], idx_v)
    plsc.subcore_barrier()

    @pl.loop(0, STEPS)
    def _(j):
        pltpu.sync_copy(ones_v, deg_sp.at[idx_v.at[j]], add=True)

    plsc.subcore_barrier()
    pltpu.sync_copy(deg_sp.at[pl.ds(sid * RPS, RPS)],
                    out_hbm.at[cid, pl.ds(sid * RPS, RPS)])


def _edge_body(hs_hbm, src_hbm, dst_hbm, zeros_hbm, out_hbm,
               sidx_v, didx_v, rows0, rows1, acc_sp, gsem0, gsem1):
    cid = lax.axis_index("c")
    sid = lax.axis_index("s")
    pltpu.sync_copy(zeros_hbm.at[pl.ds(sid * RPS, RPS)],
                    acc_sp.at[pl.ds(sid * RPS, RPS)])
    plsc.subcore_barrier()

    base = jnp.where(cid == 0, sid * S_A, NS * S_A + sid * S_B)
    nph = jnp.where(cid == 0, S_A // PH, S_B // PH)

    @pl.loop(0, nph)
    def _(ph):
        pltpu.sync_copy(src_hbm.at[pl.ds(base + ph * PH, PH)], sidx_v)
        pltpu.sync_copy(dst_hbm.at[pl.ds(base + ph * PH, PH)], didx_v)

        @pl.loop(0, PH, step=2)
        def _(j):
            da = pltpu.async_copy(hs_hbm.at[sidx_v.at[j]], rows0, gsem0)
            db = pltpu.async_copy(hs_hbm.at[sidx_v.at[j + 1]], rows1, gsem1)
            da.wait()
            pltpu.sync_copy(rows0, acc_sp.at[didx_v.at[j]], add=True)
            db.wait()
            pltpu.sync_copy(rows1, acc_sp.at[didx_v.at[j + 1]], add=True)

    plsc.subcore_barrier()
    pltpu.sync_copy(acc_sp.at[pl.ds(sid * RPS, RPS)],
                    out_hbm.at[cid, pl.ds(sid * RPS, RPS)])


@functools.cache
def _sc_kernels():
    mesh = plsc.VectorSubcoreMesh(core_axis_name="c", subcore_axis_name="s",
                                  num_cores=NC, num_subcores=NS)
    deg_kernel = pl.kernel(
        _deg_body,
        out_type=jax.ShapeDtypeStruct((NC, NP, D), jnp.float32),
        mesh=mesh,
        scratch_types=[
            pltpu.VMEM((STEPS, K), jnp.int32),
            pltpu.VMEM((K, D), jnp.float32),
            pltpu.VMEM_SHARED((NP, D), jnp.float32),
            pltpu.SemaphoreType.DMA,
        ],
    )
    edge_kernel = pl.kernel(
        _edge_body,
        out_type=jax.ShapeDtypeStruct((NC, NP, D), jnp.float32),
        mesh=mesh,
        scratch_types=[
            pltpu.VMEM((PH, K), jnp.int32),
            pltpu.VMEM((PH, K), jnp.int32),
            pltpu.VMEM((K, D), jnp.float32),
            pltpu.VMEM((K, D), jnp.float32),
            pltpu.VMEM_SHARED((NP, D), jnp.float32),
            pltpu.SemaphoreType.DMA,
            pltpu.SemaphoreType.DMA,
        ],
    )
    return deg_kernel, edge_kernel


BH = 640


def _hs_body(x_ref, w_ref, d0_ref, d1_ref, hs_ref):
    deg = d0_ref[...] + d1_ref[...] + 1.0
    dinv = lax.rsqrt(deg)
    hs_ref[...] = jnp.dot(x_ref[...], w_ref[...],
                          preferred_element_type=jnp.float32) * dinv


_hs_call = pl.pallas_call(
    _hs_body,
    grid=(NP // BH,),
    in_specs=[
        pl.BlockSpec((BH, D), lambda i: (i, 0)),
        pl.BlockSpec((D, D), lambda i: (0, 0)),
        pl.BlockSpec((BH, 1), lambda i: (i, 0)),
        pl.BlockSpec((BH, 1), lambda i: (i, 0)),
    ],
    out_specs=pl.BlockSpec((BH, D), lambda i: (i, 0)),
    out_shape=jax.ShapeDtypeStruct((NP, D), jnp.float32),
)


def _out_body(acc0_ref, acc1_ref, hs_ref, d0_ref, d1_ref, b_ref, o_ref):
    deg = d0_ref[...] + d1_ref[...] + 1.0
    dinv = lax.rsqrt(deg)
    o_ref[...] = dinv * (acc0_ref[...] + acc1_ref[...] + hs_ref[...]) + b_ref[...]


_out_call = pl.pallas_call(
    _out_body,
    grid=(NP // BH,),
    in_specs=[
        pl.BlockSpec((BH, D), lambda i: (i, 0)),
        pl.BlockSpec((BH, D), lambda i: (i, 0)),
        pl.BlockSpec((BH, D), lambda i: (i, 0)),
        pl.BlockSpec((BH, 1), lambda i: (i, 0)),
        pl.BlockSpec((BH, 1), lambda i: (i, 0)),
        pl.BlockSpec((1, D), lambda i: (0, 0)),
    ],
    out_specs=pl.BlockSpec((BH, D), lambda i: (i, 0)),
    out_shape=jax.ShapeDtypeStruct((NP, D), jnp.float32),
)


def kernel(x, edge_index, W, b):
    src = edge_index[0].astype(jnp.int32)
    dst = edge_index[1].astype(jnp.int32)
    pad = jnp.full((EP - E,), N, jnp.int32)
    src3 = jnp.concatenate([src, pad]).reshape(TOT, K)
    dst3 = jnp.concatenate([dst, pad]).reshape(TOT, K)

    ones128 = jnp.ones((K, D), jnp.float32)
    zerosD = jnp.zeros((NP, D), jnp.float32)

    deg_kernel, edge_kernel = _sc_kernels()
    degp = deg_kernel(dst3, ones128, zerosD)
    d0 = degp[0][:, 0:1]
    d1 = degp[1][:, 0:1]

    x_pad = jnp.concatenate([x, jnp.zeros((NP - N, D), x.dtype)])
    hs = _hs_call(x_pad, W, d0, d1)

    accp = edge_kernel(hs, src3, dst3, zerosD)

    out_pad = _out_call(accp[0], accp[1], hs, d0, d1, b.reshape(1, D))
    return out_pad[:N]
